# baseline (speedup 1.0000x reference)
import functools

import jax
import jax.numpy as jnp
from jax import lax
from jax.experimental import pallas as pl
from jax.experimental.pallas import tpu as pltpu
from jax.experimental.pallas import tpu_sc as plsc

N_PAD = 10240
LANES = 16
N_TILES = 32


def _tile_id():
    return lax.axis_index("s") * 2 + lax.axis_index("c")


def _zero_fill(ref, n):
    zeros = jnp.zeros((LANES,), jnp.float32)

    def body(i, c):
        ref[pl.ds(i * LANES, LANES)] = zeros
        return c

    lax.fori_loop(0, n // LANES, body, 0)


def _make_deg(ne):
    ept = ne // N_TILES
    mesh = plsc.VectorSubcoreMesh(core_axis_name="c", subcore_axis_name="s")

    @functools.partial(
        pl.kernel,
        out_type=jax.ShapeDtypeStruct((N_TILES, N_PAD), jnp.float32),
        mesh=mesh,
        compiler_params=pltpu.CompilerParams(needs_layout_passes=False),
        scratch_types=[
            pltpu.VMEM((ept,), jnp.int32),
            pltpu.VMEM((ept,), jnp.float32),
            pltpu.VMEM((N_PAD,), jnp.float32),
        ],
    )
    def deg_k(dst_hbm, ew_hbm, out_hbm, dst_v, ew_v, acc_v):
        tid = _tile_id()
        pltpu.sync_copy(dst_hbm.at[pl.ds(tid * ept, ept)], dst_v)
        pltpu.sync_copy(ew_hbm.at[pl.ds(tid * ept, ept)], ew_v)
        _zero_fill(acc_v, N_PAD)

        def body(i, c):
            b = i * LANES
            dv = dst_v[pl.ds(b, LANES)]
            wv = ew_v[pl.ds(b, LANES)]
            plsc.addupdate_scatter(acc_v, [dv], wv)
            return c

        lax.fori_loop(0, ept // LANES, body, 0)
        pltpu.sync_copy(acc_v, out_hbm.at[tid])

    return deg_k


def _make_agg(ne, n_feat, chunk, fpt=2, unroll=2):
    groups = n_feat // fpt
    n_parts = N_TILES // groups
    epp = ne // n_parts
    nchunks = epp // chunk
    assert epp % chunk == 0 and chunk % (LANES * unroll) == 0
    mesh = plsc.VectorSubcoreMesh(core_axis_name="c", subcore_axis_name="s")

    @functools.partial(
        pl.kernel,
        out_type=jax.ShapeDtypeStruct((N_TILES, fpt, N_PAD), jnp.float32),
        mesh=mesh,
        compiler_params=pltpu.CompilerParams(needs_layout_passes=False),
        scratch_types=(
            [pltpu.VMEM((N_PAD,), jnp.float32) for _ in range(2 * fpt)] + [
                pltpu.VMEM((2, chunk), jnp.int32),
                pltpu.VMEM((2, chunk), jnp.int32),
                pltpu.VMEM((2, chunk), jnp.float32),
                pltpu.SemaphoreType.DMA,
                pltpu.SemaphoreType.DMA,
            ]
        ),
    )
    def agg_k(tbl_hbm, src_hbm, dst_hbm, ew_hbm, out_hbm, *rest):
        tables = rest[0:fpt]
        accs = rest[fpt:2 * fpt]
        src_v, dst_v, ew_v, sem0, sem1 = rest[2 * fpt:]
        tid = _tile_id()
        group = lax.rem(tid, groups)
        part = tid // groups
        base_e = part * epp
        sems = (sem0, sem1)

        def issue(ci, buf):
            off = base_e + ci * chunk
            pltpu.async_copy(src_hbm.at[pl.ds(off, chunk)], src_v.at[buf], sems[buf])
            pltpu.async_copy(dst_hbm.at[pl.ds(off, chunk)], dst_v.at[buf], sems[buf])
            pltpu.async_copy(ew_hbm.at[pl.ds(off, chunk)], ew_v.at[buf], sems[buf])

        def drain(ci, buf):
            off = base_e + ci * chunk
            pltpu.make_async_copy(src_hbm.at[pl.ds(off, chunk)], src_v.at[buf], sems[buf]).wait()
            pltpu.make_async_copy(dst_hbm.at[pl.ds(off, chunk)], dst_v.at[buf], sems[buf]).wait()
            pltpu.make_async_copy(ew_hbm.at[pl.ds(off, chunk)], ew_v.at[buf], sems[buf]).wait()

        issue(0, 0)
        for f in range(fpt):
            pltpu.async_copy(tbl_hbm.at[group * fpt + f], tables[f], sem1)
        for f in range(fpt):
            _zero_fill(accs[f], N_PAD)
        for f in range(fpt):
            pltpu.make_async_copy(tbl_hbm.at[group * fpt + f], tables[f], sem1).wait()

        for ci in range(nchunks):
            buf = ci % 2
            if ci + 1 < nchunks:
                issue(ci + 1, 1 - buf)
            drain(ci, buf)

            def body(i, c2):
                b = i * (LANES * unroll)
                for u in range(unroll):
                    o = b + u * LANES
                    sv = src_v[buf, pl.ds(o, LANES)]
                    dv = dst_v[buf, pl.ds(o, LANES)]
                    wv = ew_v[buf, pl.ds(o, LANES)]
                    for f in range(fpt):
                        vv = plsc.load_gather(tables[f], [sv])
                        plsc.addupdate_scatter(accs[f], [dv], vv * wv)
                return c2

            lax.fori_loop(0, chunk // (LANES * unroll), body, 0)

        for f in range(fpt):
            pltpu.sync_copy(accs[f], out_hbm.at[tid, f])

    return agg_k


def _reduce_parts(a, d):
    n_tiles, fpt, n = a.shape
    groups = d // fpt
    r = a[0:groups]
    for p in range(1, n_tiles // groups):
        r = r + a[p * groups:(p + 1) * groups]
    return r.reshape(d, n)


def _tc1_body(xt_ref, w1t_ref, degp_ref, y1t_ref, dinv_ref):
    deg = jnp.sum(degp_ref[...], axis=0, keepdims=True) + 1.0
    dinv = lax.rsqrt(deg)
    y1t = jnp.dot(w1t_ref[...], xt_ref[...], preferred_element_type=jnp.float32)
    y1t_ref[...] = y1t * dinv
    dinv_ref[...] = dinv


def _tc2_body(y1t_ref, agg1_ref, dinv_ref, b1_ref, w2t_ref, y2t_ref):
    dinv = dinv_ref[...]
    agg1 = _reduce_parts(agg1_ref[...], y1t_ref.shape[0])
    ht = jnp.maximum(dinv * (agg1 + y1t_ref[...]) + b1_ref[...], 0.0)
    y2t = jnp.dot(w2t_ref[...], ht, preferred_element_type=jnp.float32)
    y2t_ref[...] = y2t * dinv


def _tc3_body(y2t_ref, agg2p_ref, dinv_ref, b2_ref, out_ref):
    agg2 = _reduce_parts(agg2p_ref[...], y2t_ref.shape[0])
    z = dinv_ref[...] * (agg2 + y2t_ref[...]) + b2_ref[...]
    m = jnp.max(z, axis=0, keepdims=True)
    e = jnp.exp(z - m)
    out_ref[...] = e / jnp.sum(e, axis=0, keepdims=True)


def kernel(x, edge_index, edge_weight, W1, b1, W2, b2):
    n_nodes, _ = x.shape
    ne = edge_weight.shape[0]
    d_hidden = W1.shape[1]
    d_out = W2.shape[1]

    ne_pad = -(-ne // (N_TILES * 10240)) * (N_TILES * 10240)
    src = jnp.pad(edge_index[0].astype(jnp.int32), (0, ne_pad - ne))
    dst = jnp.pad(edge_index[1].astype(jnp.int32), (0, ne_pad - ne))
    ew = jnp.pad(edge_weight.astype(jnp.float32), (0, ne_pad - ne))
    xt = jnp.pad(x.T, ((0, 0), (0, N_PAD - n_nodes)))

    degp = _make_deg(ne_pad)(dst, ew)

    y1t, dinv = pl.pallas_call(
        _tc1_body,
        out_shape=(jax.ShapeDtypeStruct((d_hidden, N_PAD), jnp.float32),
                   jax.ShapeDtypeStruct((1, N_PAD), jnp.float32)),
    )(xt, W1.T, degp)

    agg1 = _make_agg(ne_pad, d_hidden, 4096, fpt=4)(y1t, src, dst, ew)

    y2t = pl.pallas_call(
        _tc2_body,
        out_shape=jax.ShapeDtypeStruct((d_out, N_PAD), jnp.float32),
    )(y1t, agg1, dinv, b1.reshape(-1, 1), W2.T)

    agg2p = _make_agg(ne_pad, d_out, 4096, fpt=4)(y2t, src, dst, ew)

    pt = pl.pallas_call(
        _tc3_body,
        out_shape=jax.ShapeDtypeStruct((d_out, N_PAD), jnp.float32),
    )(y2t, agg2p, dinv, b2.reshape(-1, 1))

    return pt[:, :n_nodes].T


# speedup vs baseline: 1.0037x; 1.0037x over previous
import functools

import jax
import jax.numpy as jnp
from jax import lax
from jax.experimental import pallas as pl
from jax.experimental.pallas import tpu as pltpu
from jax.experimental.pallas import tpu_sc as plsc

N_PAD = 10240
LANES = 16
N_TILES = 32


def _tile_id():
    return lax.axis_index("s") * 2 + lax.axis_index("c")


def _zero_fill(ref, n):
    zeros = jnp.zeros((LANES,), jnp.float32)

    def body(i, c):
        ref[pl.ds(i * LANES, LANES)] = zeros
        return c

    lax.fori_loop(0, n // LANES, body, 0)


def _make_deg(ne):
    ept = ne // N_TILES
    mesh = plsc.VectorSubcoreMesh(core_axis_name="c", subcore_axis_name="s")

    @functools.partial(
        pl.kernel,
        out_type=jax.ShapeDtypeStruct((N_TILES, N_PAD), jnp.float32),
        mesh=mesh,
        compiler_params=pltpu.CompilerParams(needs_layout_passes=False),
        scratch_types=[
            pltpu.VMEM((ept,), jnp.int32),
            pltpu.VMEM((ept,), jnp.float32),
            pltpu.VMEM((N_PAD,), jnp.float32),
        ],
    )
    def deg_k(dst_hbm, ew_hbm, out_hbm, dst_v, ew_v, acc_v):
        tid = _tile_id()
        pltpu.sync_copy(dst_hbm.at[pl.ds(tid * ept, ept)], dst_v)
        pltpu.sync_copy(ew_hbm.at[pl.ds(tid * ept, ept)], ew_v)
        _zero_fill(acc_v, N_PAD)

        def body(i, c):
            b = i * LANES
            dv = dst_v[pl.ds(b, LANES)]
            wv = ew_v[pl.ds(b, LANES)]
            plsc.addupdate_scatter(acc_v, [dv], wv)
            return c

        lax.fori_loop(0, ept // LANES, body, 0)
        pltpu.sync_copy(acc_v, out_hbm.at[tid])

    return deg_k


def _make_agg(ne, n_feat, chunk, fpt=2, unroll=2):
    groups = n_feat // fpt
    n_parts = N_TILES // groups
    epp = ne // n_parts
    nchunks = epp // chunk
    assert epp % chunk == 0 and chunk % (LANES * unroll) == 0
    mesh = plsc.VectorSubcoreMesh(core_axis_name="c", subcore_axis_name="s")

    @functools.partial(
        pl.kernel,
        out_type=jax.ShapeDtypeStruct((N_TILES, fpt, N_PAD), jnp.float32),
        mesh=mesh,
        compiler_params=pltpu.CompilerParams(needs_layout_passes=False),
        scratch_types=(
            [pltpu.VMEM((N_PAD,), jnp.float32) for _ in range(2 * fpt)] + [
                pltpu.VMEM((2, chunk), jnp.int32),
                pltpu.VMEM((2, chunk), jnp.int32),
                pltpu.VMEM((2, chunk), jnp.float32),
                pltpu.SemaphoreType.DMA,
                pltpu.SemaphoreType.DMA,
            ]
        ),
    )
    def agg_k(tbl_hbm, src_hbm, dst_hbm, ew_hbm, out_hbm, *rest):
        tables = rest[0:fpt]
        accs = rest[fpt:2 * fpt]
        src_v, dst_v, ew_v, sem0, sem1 = rest[2 * fpt:]
        tid = _tile_id()
        group = lax.rem(tid, groups)
        part = tid // groups
        base_e = part * epp
        sems = (sem0, sem1)

        def issue(ci, buf):
            off = base_e + ci * chunk
            pltpu.async_copy(src_hbm.at[pl.ds(off, chunk)], src_v.at[buf], sems[buf])
            pltpu.async_copy(dst_hbm.at[pl.ds(off, chunk)], dst_v.at[buf], sems[buf])
            pltpu.async_copy(ew_hbm.at[pl.ds(off, chunk)], ew_v.at[buf], sems[buf])

        def drain(ci, buf):
            off = base_e + ci * chunk
            pltpu.make_async_copy(src_hbm.at[pl.ds(off, chunk)], src_v.at[buf], sems[buf]).wait()
            pltpu.make_async_copy(dst_hbm.at[pl.ds(off, chunk)], dst_v.at[buf], sems[buf]).wait()
            pltpu.make_async_copy(ew_hbm.at[pl.ds(off, chunk)], ew_v.at[buf], sems[buf]).wait()

        issue(0, 0)
        for f in range(fpt):
            pltpu.async_copy(tbl_hbm.at[group * fpt + f], tables[f], sem1)
        for f in range(fpt):
            _zero_fill(accs[f], N_PAD)
        for f in range(fpt):
            pltpu.make_async_copy(tbl_hbm.at[group * fpt + f], tables[f], sem1).wait()

        for ci in range(nchunks):
            buf = ci % 2
            if ci + 1 < nchunks:
                issue(ci + 1, 1 - buf)
            drain(ci, buf)

            def body(i, c2):
                b = i * (LANES * unroll)
                for u in range(unroll):
                    o = b + u * LANES
                    sv = src_v[buf, pl.ds(o, LANES)]
                    dv = dst_v[buf, pl.ds(o, LANES)]
                    wv = ew_v[buf, pl.ds(o, LANES)]
                    for f in range(fpt):
                        vv = plsc.load_gather(tables[f], [sv])
                        plsc.addupdate_scatter(accs[f], [dv], vv * wv)
                return c2

            lax.fori_loop(0, chunk // (LANES * unroll), body, 0)

        for f in range(fpt):
            pltpu.sync_copy(accs[f], out_hbm.at[tid, f])

    return agg_k


def _reduce_parts(a, d):
    n_tiles, fpt, n = a.shape
    groups = d // fpt
    r = a[0:groups]
    for p in range(1, n_tiles // groups):
        r = r + a[p * groups:(p + 1) * groups]
    return r.reshape(d, n)


def _tc1_body(xt_ref, w1t_ref, degp_ref, y1t_ref, dinv_ref):
    deg = jnp.sum(degp_ref[...], axis=0, keepdims=True) + 1.0
    dinv = lax.rsqrt(deg)
    y1t = jnp.dot(w1t_ref[...], xt_ref[...], preferred_element_type=jnp.float32)
    y1t_ref[...] = y1t * dinv
    dinv_ref[...] = dinv


def _tc2_body(y1t_ref, agg1_ref, dinv_ref, b1_ref, w2t_ref, y2t_ref):
    dinv = dinv_ref[...]
    agg1 = _reduce_parts(agg1_ref[...], y1t_ref.shape[0])
    ht = jnp.maximum(dinv * (agg1 + y1t_ref[...]) + b1_ref[...], 0.0)
    y2t = jnp.dot(w2t_ref[...], ht, preferred_element_type=jnp.float32)
    y2t_ref[...] = y2t * dinv


def _tc3_body(y2t_ref, agg2p_ref, dinv_ref, b2_ref, out_ref):
    agg2 = _reduce_parts(agg2p_ref[...], y2t_ref.shape[0])
    z = dinv_ref[...] * (agg2 + y2t_ref[...]) + b2_ref[...]
    m = jnp.max(z, axis=0, keepdims=True)
    e = jnp.exp(z - m)
    out_ref[...] = e / jnp.sum(e, axis=0, keepdims=True)


def kernel(x, edge_index, edge_weight, W1, b1, W2, b2):
    n_nodes, _ = x.shape
    ne = edge_weight.shape[0]
    d_hidden = W1.shape[1]
    d_out = W2.shape[1]

    ne_pad = -(-ne // (N_TILES * 10240)) * (N_TILES * 10240)
    src = jnp.pad(edge_index[0].astype(jnp.int32), (0, ne_pad - ne))
    dst = jnp.pad(edge_index[1].astype(jnp.int32), (0, ne_pad - ne))
    ew = jnp.pad(edge_weight.astype(jnp.float32), (0, ne_pad - ne))
    xt = jnp.pad(x.T, ((0, 0), (0, N_PAD - n_nodes)))

    degp = _make_deg(ne_pad)(dst, ew)

    y1t, dinv = pl.pallas_call(
        _tc1_body,
        out_shape=(jax.ShapeDtypeStruct((d_hidden, N_PAD), jnp.float32),
                   jax.ShapeDtypeStruct((1, N_PAD), jnp.float32)),
    )(xt, W1.T, degp)

    agg1 = _make_agg(ne_pad, d_hidden, 4096, fpt=4)(y1t, src, dst, ew)

    y2t = pl.pallas_call(
        _tc2_body,
        out_shape=jax.ShapeDtypeStruct((d_out, N_PAD), jnp.float32),
    )(y1t, agg1, dinv, b1.reshape(-1, 1), W2.T)

    agg2p = _make_agg(ne_pad, d_out, 4096, fpt=2)(y2t, src, dst, ew)

    pt = pl.pallas_call(
        _tc3_body,
        out_shape=jax.ShapeDtypeStruct((d_out, N_PAD), jnp.float32),
    )(y2t, agg2p, dinv, b2.reshape(-1, 1))

    return pt[:, :n_nodes].T


# speedup vs baseline: 1.1276x; 1.1235x over previous
import functools

import jax
import jax.numpy as jnp
from jax import lax
from jax.experimental import pallas as pl
from jax.experimental.pallas import tpu as pltpu
from jax.experimental.pallas import tpu_sc as plsc

N_PAD = 10240
LANES = 16
N_TILES = 32


def _tile_id():
    return lax.axis_index("s") * 2 + lax.axis_index("c")


def _zero_fill(ref, n):
    zeros = jnp.zeros((LANES,), jnp.float32)

    def body(i, c):
        ref[pl.ds(i * LANES, LANES)] = zeros
        return c

    lax.fori_loop(0, n // LANES, body, 0)


def _make_deg(ne):
    ept = ne // N_TILES
    mesh = plsc.VectorSubcoreMesh(core_axis_name="c", subcore_axis_name="s")

    @functools.partial(
        pl.kernel,
        out_type=jax.ShapeDtypeStruct((N_TILES, N_PAD), jnp.float32),
        mesh=mesh,
        compiler_params=pltpu.CompilerParams(needs_layout_passes=False),
        scratch_types=[
            pltpu.VMEM((ept,), jnp.int32),
            pltpu.VMEM((ept,), jnp.float32),
            pltpu.VMEM((N_PAD,), jnp.float32),
        ],
    )
    def deg_k(dst_hbm, ew_hbm, out_hbm, dst_v, ew_v, acc_v):
        tid = _tile_id()
        pltpu.sync_copy(dst_hbm.at[pl.ds(tid * ept, ept)], dst_v)
        pltpu.sync_copy(ew_hbm.at[pl.ds(tid * ept, ept)], ew_v)
        _zero_fill(acc_v, N_PAD)

        def body(i, c):
            b = i * LANES
            dv = dst_v[pl.ds(b, LANES)]
            wv = ew_v[pl.ds(b, LANES)]
            plsc.addupdate_scatter(acc_v, [dv], wv)
            return c

        lax.fori_loop(0, ept // LANES, body, 0)
        pltpu.sync_copy(acc_v, out_hbm.at[tid])

    return deg_k


def _make_agg(ne, n_feat, chunk, fpt=2, unroll=2):
    groups = n_feat // fpt
    n_parts = N_TILES // groups
    epp = ne // n_parts
    nchunks = epp // chunk
    assert epp % chunk == 0 and chunk % (LANES * unroll) == 0
    mesh = plsc.VectorSubcoreMesh(core_axis_name="c", subcore_axis_name="s")

    @functools.partial(
        pl.kernel,
        out_type=jax.ShapeDtypeStruct((N_TILES, fpt, N_PAD), jnp.float32),
        mesh=mesh,
        compiler_params=pltpu.CompilerParams(needs_layout_passes=False),
        scratch_types=(
            [pltpu.VMEM((N_PAD,), jnp.float32) for _ in range(2 * fpt)] + [
                pltpu.VMEM((2, chunk), jnp.int32),
                pltpu.VMEM((2, chunk), jnp.int32),
                pltpu.VMEM((2, chunk), jnp.float32),
                pltpu.SemaphoreType.DMA,
                pltpu.SemaphoreType.DMA,
            ]
        ),
    )
    def agg_k(tbl_hbm, src_hbm, dst_hbm, ew_hbm, out_hbm, *rest):
        tables = rest[0:fpt]
        accs = rest[fpt:2 * fpt]
        src_v, dst_v, ew_v, sem0, sem1 = rest[2 * fpt:]
        tid = _tile_id()
        group = lax.rem(tid, groups)
        part = tid // groups
        base_e = part * epp
        sems = (sem0, sem1)

        def issue(ci, buf):
            off = base_e + ci * chunk
            pltpu.async_copy(src_hbm.at[pl.ds(off, chunk)], src_v.at[buf], sems[buf])
            pltpu.async_copy(dst_hbm.at[pl.ds(off, chunk)], dst_v.at[buf], sems[buf])
            pltpu.async_copy(ew_hbm.at[pl.ds(off, chunk)], ew_v.at[buf], sems[buf])

        def drain(ci, buf):
            off = base_e + ci * chunk
            pltpu.make_async_copy(src_hbm.at[pl.ds(off, chunk)], src_v.at[buf], sems[buf]).wait()
            pltpu.make_async_copy(dst_hbm.at[pl.ds(off, chunk)], dst_v.at[buf], sems[buf]).wait()
            pltpu.make_async_copy(ew_hbm.at[pl.ds(off, chunk)], ew_v.at[buf], sems[buf]).wait()

        issue(0, 0)
        for f in range(fpt):
            pltpu.sync_copy(tbl_hbm.at[group * fpt + f], tables[f])
            _zero_fill(accs[f], N_PAD)

        for ci in range(nchunks):
            buf = ci % 2
            if ci + 1 < nchunks:
                issue(ci + 1, 1 - buf)
            drain(ci, buf)

            def body(i, c2):
                b = i * (LANES * unroll)
                for u in range(unroll):
                    o = b + u * LANES
                    sv = src_v[buf, pl.ds(o, LANES)]
                    dv = dst_v[buf, pl.ds(o, LANES)]
                    wv = ew_v[buf, pl.ds(o, LANES)]
                    for f in range(fpt):
                        vv = plsc.load_gather(tables[f], [sv])
                        plsc.addupdate_scatter(accs[f], [dv], vv * wv)
                return c2

            lax.fori_loop(0, chunk // (LANES * unroll), body, 0)

        for f in range(fpt):
            pltpu.sync_copy(accs[f], out_hbm.at[tid, f])

    return agg_k


def _reduce_parts(a, d):
    n_tiles, fpt, n = a.shape
    groups = d // fpt
    r = a[0:groups]
    for p in range(1, n_tiles // groups):
        r = r + a[p * groups:(p + 1) * groups]
    return r.reshape(d, n)


def _tc1_body(xt_ref, w1t_ref, degp_ref, y1t_ref, dinv_ref):
    deg = jnp.sum(degp_ref[...], axis=0, keepdims=True) + 1.0
    dinv = lax.rsqrt(deg)
    y1t = jnp.dot(w1t_ref[...], xt_ref[...], preferred_element_type=jnp.float32)
    y1t_ref[...] = y1t * dinv
    dinv_ref[...] = dinv


def _tc2_body(y1t_ref, agg1_ref, dinv_ref, b1_ref, w2t_ref, y2t_ref):
    dinv = dinv_ref[...]
    agg1 = _reduce_parts(agg1_ref[...], y1t_ref.shape[0])
    ht = jnp.maximum(dinv * (agg1 + y1t_ref[...]) + b1_ref[...], 0.0)
    y2t = jnp.dot(w2t_ref[...], ht, preferred_element_type=jnp.float32)
    y2t_ref[...] = y2t * dinv


def _tc3_body(y2t_ref, agg2p_ref, dinv_ref, b2_ref, out_ref):
    agg2 = _reduce_parts(agg2p_ref[...], y2t_ref.shape[0])
    z = dinv_ref[...] * (agg2 + y2t_ref[...]) + b2_ref[...]
    m = jnp.max(z, axis=0, keepdims=True)
    e = jnp.exp(z - m)
    out_ref[...] = e / jnp.sum(e, axis=0, keepdims=True)


def kernel(x, edge_index, edge_weight, W1, b1, W2, b2):
    n_nodes, _ = x.shape
    ne = edge_weight.shape[0]
    d_hidden = W1.shape[1]
    d_out = W2.shape[1]

    src = edge_index[0].astype(jnp.int32)
    dst = edge_index[1].astype(jnp.int32)
    ew = edge_weight.astype(jnp.float32)
    xt = jnp.pad(x.T, ((0, 0), (0, N_PAD - n_nodes)))

    degp = _make_deg(ne)(dst, ew)

    y1t, dinv = pl.pallas_call(
        _tc1_body,
        out_shape=(jax.ShapeDtypeStruct((d_hidden, N_PAD), jnp.float32),
                   jax.ShapeDtypeStruct((1, N_PAD), jnp.float32)),
    )(xt, W1.T, degp)

    agg1 = _make_agg(ne, d_hidden, 3200, fpt=4)(y1t, src, dst, ew)

    y2t = pl.pallas_call(
        _tc2_body,
        out_shape=jax.ShapeDtypeStruct((d_out, N_PAD), jnp.float32),
    )(y1t, agg1, dinv, b1.reshape(-1, 1), W2.T)

    agg2p = _make_agg(ne, d_out, 3200, fpt=2)(y2t, src, dst, ew)

    pt = pl.pallas_call(
        _tc3_body,
        out_shape=jax.ShapeDtypeStruct((d_out, N_PAD), jnp.float32),
    )(y2t, agg2p, dinv, b2.reshape(-1, 1))

    return pt[:, :n_nodes].T


# speedup vs baseline: 1.1517x; 1.0214x over previous
import functools

import jax
import jax.numpy as jnp
from jax import lax
from jax.experimental import pallas as pl
from jax.experimental.pallas import tpu as pltpu
from jax.experimental.pallas import tpu_sc as plsc

N_PAD = 10240
LANES = 16
N_TILES = 32


def _tile_id():
    return lax.axis_index("s") * 2 + lax.axis_index("c")


def _zero_fill(ref, n):
    zeros = jnp.zeros((LANES,), jnp.float32)

    def body(i, c):
        ref[pl.ds(i * LANES, LANES)] = zeros
        return c

    lax.fori_loop(0, n // LANES, body, 0)


def _make_deg(ne):
    ept = ne // N_TILES
    mesh = plsc.VectorSubcoreMesh(core_axis_name="c", subcore_axis_name="s")

    @functools.partial(
        pl.kernel,
        out_type=jax.ShapeDtypeStruct((N_TILES, N_PAD), jnp.float32),
        mesh=mesh,
        compiler_params=pltpu.CompilerParams(needs_layout_passes=False),
        scratch_types=[
            pltpu.VMEM((ept,), jnp.int32),
            pltpu.VMEM((ept,), jnp.float32),
            pltpu.VMEM((N_PAD,), jnp.float32),
        ],
    )
    def deg_k(dst_hbm, ew_hbm, out_hbm, dst_v, ew_v, acc_v):
        tid = _tile_id()
        pltpu.sync_copy(dst_hbm.at[pl.ds(tid * ept, ept)], dst_v)
        pltpu.sync_copy(ew_hbm.at[pl.ds(tid * ept, ept)], ew_v)
        _zero_fill(acc_v, N_PAD)

        def body(i, c):
            b = i * LANES
            dv = dst_v[pl.ds(b, LANES)]
            wv = ew_v[pl.ds(b, LANES)]
            plsc.addupdate_scatter(acc_v, [dv], wv)
            return c

        lax.fori_loop(0, ept // LANES, body, 0)
        pltpu.sync_copy(acc_v, out_hbm.at[tid])

    return deg_k


def _make_agg(ne, n_feat, chunk, fpt=2, unroll=2):
    groups = n_feat // fpt
    n_parts = N_TILES // groups
    epp = ne // n_parts
    nchunks = epp // chunk
    assert epp % chunk == 0 and chunk % (LANES * unroll) == 0
    mesh = plsc.VectorSubcoreMesh(core_axis_name="c", subcore_axis_name="s")

    @functools.partial(
        pl.kernel,
        out_type=jax.ShapeDtypeStruct((N_TILES, fpt, N_PAD), jnp.float32),
        mesh=mesh,
        compiler_params=pltpu.CompilerParams(needs_layout_passes=False),
        scratch_types=(
            [pltpu.VMEM((N_PAD,), jnp.float32) for _ in range(2 * fpt)] + [
                pltpu.VMEM((2, chunk), jnp.int32),
                pltpu.VMEM((2, chunk), jnp.int32),
                pltpu.VMEM((2, chunk), jnp.float32),
                pltpu.SemaphoreType.DMA,
                pltpu.SemaphoreType.DMA,
            ]
        ),
    )
    def agg_k(tbl_hbm, src_hbm, dst_hbm, ew_hbm, out_hbm, *rest):
        tables = rest[0:fpt]
        accs = rest[fpt:2 * fpt]
        src_v, dst_v, ew_v, sem0, sem1 = rest[2 * fpt:]
        tid = _tile_id()
        group = lax.rem(tid, groups)
        part = tid // groups
        base_e = part * epp
        sems = (sem0, sem1)

        def issue(ci, buf):
            off = base_e + ci * chunk
            pltpu.async_copy(src_hbm.at[pl.ds(off, chunk)], src_v.at[buf], sems[buf])
            pltpu.async_copy(dst_hbm.at[pl.ds(off, chunk)], dst_v.at[buf], sems[buf])
            pltpu.async_copy(ew_hbm.at[pl.ds(off, chunk)], ew_v.at[buf], sems[buf])

        def drain(ci, buf):
            off = base_e + ci * chunk
            pltpu.make_async_copy(src_hbm.at[pl.ds(off, chunk)], src_v.at[buf], sems[buf]).wait()
            pltpu.make_async_copy(dst_hbm.at[pl.ds(off, chunk)], dst_v.at[buf], sems[buf]).wait()
            pltpu.make_async_copy(ew_hbm.at[pl.ds(off, chunk)], ew_v.at[buf], sems[buf]).wait()

        issue(0, 0)
        for f in range(fpt):
            pltpu.sync_copy(tbl_hbm.at[group * fpt + f], tables[f])
            _zero_fill(accs[f], N_PAD)

        for ci in range(nchunks):
            buf = ci % 2
            if ci + 1 < nchunks:
                issue(ci + 1, 1 - buf)
            drain(ci, buf)

            def body(i, c2):
                b = i * (LANES * unroll)
                for u in range(unroll):
                    o = b + u * LANES
                    sv = src_v[buf, pl.ds(o, LANES)]
                    dv = dst_v[buf, pl.ds(o, LANES)]
                    wv = ew_v[buf, pl.ds(o, LANES)]
                    for f in range(fpt):
                        vv = plsc.load_gather(tables[f], [sv])
                        plsc.addupdate_scatter(accs[f], [dv], vv * wv)
                return c2

            lax.fori_loop(0, chunk // (LANES * unroll), body, 0)

        for f in range(fpt):
            pltpu.sync_copy(accs[f], out_hbm.at[tid, f])

    return agg_k


def _reduce_parts(a, d):
    n_tiles, fpt, n = a.shape
    groups = d // fpt
    r = a[0:groups]
    for p in range(1, n_tiles // groups):
        r = r + a[p * groups:(p + 1) * groups]
    return r.reshape(d, n)


def _tc1_body(xt_ref, w1t_ref, degp_ref, y1t_ref, dinv_ref):
    deg = jnp.sum(degp_ref[...], axis=0, keepdims=True) + 1.0
    dinv = lax.rsqrt(deg)
    y1t = jnp.dot(w1t_ref[...], xt_ref[...], preferred_element_type=jnp.float32)
    y1t_ref[...] = y1t * dinv
    dinv_ref[...] = dinv


def _tc2_body(y1t_ref, agg1_ref, dinv_ref, b1_ref, w2t_ref, y2t_ref):
    dinv = dinv_ref[...]
    agg1 = _reduce_parts(agg1_ref[...], y1t_ref.shape[0])
    ht = jnp.maximum(dinv * (agg1 + y1t_ref[...]) + b1_ref[...], 0.0)
    y2t = jnp.dot(w2t_ref[...], ht, preferred_element_type=jnp.float32)
    y2t_ref[...] = y2t * dinv


def _tc3_body(y2t_ref, agg2p_ref, dinv_ref, b2_ref, out_ref):
    agg2 = _reduce_parts(agg2p_ref[...], y2t_ref.shape[0])
    z = dinv_ref[...] * (agg2 + y2t_ref[...]) + b2_ref[...]
    m = jnp.max(z, axis=0, keepdims=True)
    e = jnp.exp(z - m)
    out_ref[...] = e / jnp.sum(e, axis=0, keepdims=True)


def kernel(x, edge_index, edge_weight, W1, b1, W2, b2):
    n_nodes, _ = x.shape
    ne = edge_weight.shape[0]
    d_hidden = W1.shape[1]
    d_out = W2.shape[1]

    ne_pad = -(-ne // (N_TILES * 10240)) * (N_TILES * 10240)
    spread = jnp.arange(ne_pad - ne, dtype=jnp.int32) % n_nodes
    src = jnp.concatenate([edge_index[0].astype(jnp.int32), spread])
    dst = jnp.concatenate([edge_index[1].astype(jnp.int32), spread])
    ew = jnp.pad(edge_weight.astype(jnp.float32), (0, ne_pad - ne))
    xt = jnp.pad(x.T, ((0, 0), (0, N_PAD - n_nodes)))

    degp = _make_deg(ne_pad)(dst, ew)

    y1t, dinv = pl.pallas_call(
        _tc1_body,
        out_shape=(jax.ShapeDtypeStruct((d_hidden, N_PAD), jnp.float32),
                   jax.ShapeDtypeStruct((1, N_PAD), jnp.float32)),
    )(xt, W1.T, degp)

    agg1 = _make_agg(ne_pad, d_hidden, 5120, fpt=4)(y1t, src, dst, ew)

    y2t = pl.pallas_call(
        _tc2_body,
        out_shape=jax.ShapeDtypeStruct((d_out, N_PAD), jnp.float32),
    )(y1t, agg1, dinv, b1.reshape(-1, 1), W2.T)

    agg2p = _make_agg(ne_pad, d_out, 5120, fpt=4)(y2t, src, dst, ew)

    pt = pl.pallas_call(
        _tc3_body,
        out_shape=jax.ShapeDtypeStruct((d_out, N_PAD), jnp.float32),
    )(y2t, agg2p, dinv, b2.reshape(-1, 1))

    return pt[:, :n_nodes].T


# speedup vs baseline: 1.4292x; 1.2409x over previous
import functools

import jax
import jax.numpy as jnp
from jax import lax
from jax.experimental import pallas as pl
from jax.experimental.pallas import tpu as pltpu
from jax.experimental.pallas import tpu_sc as plsc

N_PAD = 10240
LANES = 16
N_TILES = 32


def _tile_id():
    return lax.axis_index("s") * 2 + lax.axis_index("c")


def _zero_fill(ref, n):
    zeros = jnp.zeros((LANES,), jnp.float32)

    def body(i, c):
        ref[pl.ds(i * LANES, LANES)] = zeros
        return c

    lax.fori_loop(0, n // LANES, body, 0)


def _make_deg(ne):
    ept = ne // N_TILES
    mesh = plsc.VectorSubcoreMesh(core_axis_name="c", subcore_axis_name="s")

    @functools.partial(
        pl.kernel,
        out_type=jax.ShapeDtypeStruct((N_TILES, N_PAD), jnp.float32),
        mesh=mesh,
        compiler_params=pltpu.CompilerParams(needs_layout_passes=False),
        scratch_types=[
            pltpu.VMEM((ept,), jnp.int32),
            pltpu.VMEM((ept,), jnp.float32),
            pltpu.VMEM((N_PAD,), jnp.float32),
        ],
    )
    def deg_k(dst_hbm, ew_hbm, out_hbm, dst_v, ew_v, acc_v):
        tid = _tile_id()
        pltpu.sync_copy(dst_hbm.at[pl.ds(tid * ept, ept)], dst_v)
        pltpu.sync_copy(ew_hbm.at[pl.ds(tid * ept, ept)], ew_v)
        _zero_fill(acc_v, N_PAD)

        def body(i, c):
            b = i * LANES
            dv = dst_v[pl.ds(b, LANES)]
            wv = ew_v[pl.ds(b, LANES)]
            plsc.addupdate_scatter(acc_v, [dv], wv)
            return c

        lax.fori_loop(0, ept // LANES, body, 0)
        pltpu.sync_copy(acc_v, out_hbm.at[tid])

    return deg_k


def _make_agg(ne, n_feat, chunk, fpt=2, unroll=2):
    groups = n_feat // fpt
    n_parts = N_TILES // groups
    epp = ne // n_parts
    nchunks = epp // chunk
    assert epp % chunk == 0 and chunk % (LANES * unroll) == 0
    mesh = plsc.VectorSubcoreMesh(core_axis_name="c", subcore_axis_name="s")

    npk = fpt // 2
    assert fpt % 2 == 0

    @functools.partial(
        pl.kernel,
        out_type=jax.ShapeDtypeStruct((N_TILES, fpt, N_PAD), jnp.float32),
        mesh=mesh,
        compiler_params=pltpu.CompilerParams(needs_layout_passes=False),
        scratch_types=(
            [pltpu.VMEM((N_PAD,), jnp.float32) for _ in range(npk + fpt)] + [
                pltpu.VMEM((2, chunk), jnp.int32),
                pltpu.VMEM((2, chunk), jnp.int32),
                pltpu.VMEM((2, chunk), jnp.float32),
                pltpu.SemaphoreType.DMA,
                pltpu.SemaphoreType.DMA,
            ]
        ),
    )
    def agg_k(tbl_hbm, src_hbm, dst_hbm, ew_hbm, out_hbm, *rest):
        tables = rest[0:npk]
        accs = rest[npk:npk + fpt]
        src_v, dst_v, ew_v, sem0, sem1 = rest[npk + fpt:]
        tid = _tile_id()
        group = lax.rem(tid, groups)
        part = tid // groups
        base_e = part * epp
        sems = (sem0, sem1)

        def issue(ci, buf):
            off = base_e + ci * chunk
            pltpu.async_copy(src_hbm.at[pl.ds(off, chunk)], src_v.at[buf], sems[buf])
            pltpu.async_copy(dst_hbm.at[pl.ds(off, chunk)], dst_v.at[buf], sems[buf])
            pltpu.async_copy(ew_hbm.at[pl.ds(off, chunk)], ew_v.at[buf], sems[buf])

        def drain(ci, buf):
            off = base_e + ci * chunk
            pltpu.make_async_copy(src_hbm.at[pl.ds(off, chunk)], src_v.at[buf], sems[buf]).wait()
            pltpu.make_async_copy(dst_hbm.at[pl.ds(off, chunk)], dst_v.at[buf], sems[buf]).wait()
            pltpu.make_async_copy(ew_hbm.at[pl.ds(off, chunk)], ew_v.at[buf], sems[buf]).wait()

        issue(0, 0)
        for j in range(npk):
            pltpu.sync_copy(tbl_hbm.at[group * npk + j], tables[j])
        for f in range(fpt):
            _zero_fill(accs[f], N_PAD)

        for ci in range(nchunks):
            buf = ci % 2
            if ci + 1 < nchunks:
                issue(ci + 1, 1 - buf)
            drain(ci, buf)

            def body(i, c2):
                b = i * (LANES * unroll)
                for u in range(unroll):
                    o = b + u * LANES
                    sv = src_v[buf, pl.ds(o, LANES)]
                    dv = dst_v[buf, pl.ds(o, LANES)]
                    wv = ew_v[buf, pl.ds(o, LANES)]
                    for j in range(npk):
                        pv = plsc.load_gather(tables[j], [sv])
                        pu = plsc.bitcast(pv, jnp.uint32)
                        va = plsc.bitcast(pu << 16, jnp.float32)
                        vb = plsc.bitcast(pu & jnp.uint32(0xFFFF0000), jnp.float32)
                        plsc.addupdate_scatter(accs[2 * j], [dv], va * wv)
                        plsc.addupdate_scatter(accs[2 * j + 1], [dv], vb * wv)
                return c2

            lax.fori_loop(0, chunk // (LANES * unroll), body, 0)

        for f in range(fpt):
            pltpu.sync_copy(accs[f], out_hbm.at[tid, f])

    return agg_k


def _pack_pairs(y):
    d, n = y.shape
    y2 = y.reshape(d // 2, 2, n)
    au = lax.bitcast_convert_type(y2[:, 0, :].astype(jnp.bfloat16), jnp.uint16)
    bu = lax.bitcast_convert_type(y2[:, 1, :].astype(jnp.bfloat16), jnp.uint16)
    packed = au.astype(jnp.uint32) | (bu.astype(jnp.uint32) << 16)
    return lax.bitcast_convert_type(packed, jnp.float32)


def _reduce_parts(a, d):
    n_tiles, fpt, n = a.shape
    groups = d // fpt
    r = a[0:groups]
    for p in range(1, n_tiles // groups):
        r = r + a[p * groups:(p + 1) * groups]
    return r.reshape(d, n)


def _tc1_body(xt_ref, w1t_ref, degp_ref, y1t_ref, y1p_ref, dinv_ref):
    deg = jnp.sum(degp_ref[...], axis=0, keepdims=True) + 1.0
    dinv = lax.rsqrt(deg)
    y1t = jnp.dot(w1t_ref[...], xt_ref[...], preferred_element_type=jnp.float32)
    y1t = y1t * dinv
    y1t_ref[...] = y1t
    y1p_ref[...] = _pack_pairs(y1t)
    dinv_ref[...] = dinv


def _tc2_body(y1t_ref, agg1_ref, dinv_ref, b1_ref, w2t_ref, y2t_ref, y2p_ref):
    dinv = dinv_ref[...]
    agg1 = _reduce_parts(agg1_ref[...], y1t_ref.shape[0])
    ht = jnp.maximum(dinv * (agg1 + y1t_ref[...]) + b1_ref[...], 0.0)
    y2t = jnp.dot(w2t_ref[...], ht, preferred_element_type=jnp.float32)
    y2t = y2t * dinv
    y2t_ref[...] = y2t
    y2p_ref[...] = _pack_pairs(y2t)


def _tc3_body(y2t_ref, agg2p_ref, dinv_ref, b2_ref, out_ref):
    agg2 = _reduce_parts(agg2p_ref[...], y2t_ref.shape[0])
    z = dinv_ref[...] * (agg2 + y2t_ref[...]) + b2_ref[...]
    m = jnp.max(z, axis=0, keepdims=True)
    e = jnp.exp(z - m)
    out_ref[...] = e / jnp.sum(e, axis=0, keepdims=True)


def kernel(x, edge_index, edge_weight, W1, b1, W2, b2):
    n_nodes, _ = x.shape
    ne = edge_weight.shape[0]
    d_hidden = W1.shape[1]
    d_out = W2.shape[1]

    ne_pad = -(-ne // (N_TILES * 10240)) * (N_TILES * 10240)
    spread = jnp.arange(ne_pad - ne, dtype=jnp.int32) % n_nodes
    src = jnp.concatenate([edge_index[0].astype(jnp.int32), spread])
    dst = jnp.concatenate([edge_index[1].astype(jnp.int32), spread])
    ew = jnp.pad(edge_weight.astype(jnp.float32), (0, ne_pad - ne))
    xt = jnp.pad(x.T, ((0, 0), (0, N_PAD - n_nodes)))

    degp = _make_deg(ne_pad)(dst, ew)

    y1t, y1p, dinv = pl.pallas_call(
        _tc1_body,
        out_shape=(jax.ShapeDtypeStruct((d_hidden, N_PAD), jnp.float32),
                   jax.ShapeDtypeStruct((d_hidden // 2, N_PAD), jnp.float32),
                   jax.ShapeDtypeStruct((1, N_PAD), jnp.float32)),
    )(xt, W1.T, degp)

    agg1 = _make_agg(ne_pad, d_hidden, 5120, fpt=4)(y1p, src, dst, ew)

    y2t, y2p = pl.pallas_call(
        _tc2_body,
        out_shape=(jax.ShapeDtypeStruct((d_out, N_PAD), jnp.float32),
                   jax.ShapeDtypeStruct((d_out // 2, N_PAD), jnp.float32)),
    )(y1t, agg1, dinv, b1.reshape(-1, 1), W2.T)

    agg2p = _make_agg(ne_pad, d_out, 5120, fpt=4)(y2p, src, dst, ew)

    pt = pl.pallas_call(
        _tc3_body,
        out_shape=jax.ShapeDtypeStruct((d_out, N_PAD), jnp.float32),
    )(y2t, agg2p, dinv, b2.reshape(-1, 1))

    return pt[:, :n_nodes].T
